# Initial kernel scaffold; baseline (speedup 1.0000x reference)
import jax
import jax.numpy as jnp
from jax.experimental import pallas as pl


def kernel(input_ids, token_type_ids, timesteps, word_emb, type_emb, W1, b1, W2, b2, gamma, beta):
    raise NotImplementedError("write your pallas kernel here")


# speedup vs baseline: 1.9280x; 1.9280x over previous
import functools
import math

import jax
import jax.numpy as jnp
from jax import lax
from jax.experimental import pallas as pl
from jax.experimental.pallas import tpu as pltpu
from jax.experimental.pallas import tpu_sc as plsc

B, T, L = 4, 2, 4096
V, D = 100000, 128
EPS = 1e-12

NC, NS = 2, 16
NW = NC * NS
N_ROWS = B * T * L
ROWS_PER_W = N_ROWS // NW
CH = 128
NCHUNK = ROWS_PER_W // CH

BLK = 2048
NBLK = (B * L) // BLK
BPB = L // BLK


def _sc_gather_body(table_hbm, idx_hbm, out_hbm, idx_v, buf0, buf1, sem0, sem1):
    wid = lax.axis_index("s") * NC + lax.axis_index("c")
    base = wid * ROWS_PER_W
    pltpu.sync_copy(idx_hbm.at[wid], idx_v)
    bufs = (buf0, buf1)
    sems = (sem0, sem1)
    handles = [None, None]
    handles[0] = pltpu.async_copy(table_hbm.at[idx_v.at[0]], bufs[0], sems[0])
    for c in range(NCHUNK):
        if c + 1 < NCHUNK:
            handles[(c + 1) % 2] = pltpu.async_copy(
                table_hbm.at[idx_v.at[c + 1]], bufs[(c + 1) % 2], sems[(c + 1) % 2])
        handles[c % 2].wait()
        pltpu.sync_copy(bufs[c % 2], out_hbm.at[pl.ds(base + c * CH, CH)])


@functools.cache
def _build_sc_gather():
    return pl.kernel(
        _sc_gather_body,
        mesh=plsc.VectorSubcoreMesh(core_axis_name="c", subcore_axis_name="s"),
        out_type=jax.ShapeDtypeStruct((N_ROWS, D), jnp.float32),
        scratch_types=[
            pltpu.VMEM((NCHUNK, CH), jnp.int32),
            pltpu.VMEM((CH, D), jnp.float32),
            pltpu.VMEM((CH, D), jnp.float32),
            pltpu.SemaphoreType.DMA,
            pltpu.SemaphoreType.DMA,
        ],
    )


def _norm_body(ts_ref, tt_ref, g_ref, type_ref, w1_ref, b1_ref, w2_ref,
               b2_ref, gamma_ref, beta_ref, out_ref):
    t_val = ts_ref[0]
    col = lax.broadcasted_iota(jnp.int32, (1, D), 1)
    is_cos = col < (D // 2)
    k = jnp.where(is_cos, col, col - D // 2).astype(jnp.float32)
    freqs = jnp.exp((-math.log(10000.0) / (D // 2)) * k)
    args = t_val * freqs
    te = jnp.where(is_cos, jnp.cos(args), jnp.sin(args))
    h = jnp.dot(te, w1_ref[...], preferred_element_type=jnp.float32) + b1_ref[...]
    h = h * jax.nn.sigmoid(h)
    trow = jnp.dot(h, w2_ref[...], preferred_element_type=jnp.float32) + b2_ref[...]

    ty0 = type_ref[0:1, :]
    dty = type_ref[1:2, :] - ty0
    base_row = ty0 + trow
    g = g_ref[...]
    tt = tt_ref[0]
    half0 = g[:, :D] + base_row + tt[:, 0:1] * dty
    half1 = g[:, D:] + base_row + tt[:, 1:2] * dty
    emb = jnp.concatenate([half0, half1], axis=-1)
    mu = jnp.mean(emb, axis=-1, keepdims=True)
    dev = emb - mu
    var = jnp.mean(dev * dev, axis=-1, keepdims=True)
    out_ref[...] = dev * lax.rsqrt(var + EPS) * gamma_ref[...] + beta_ref[...]


_norm = pl.pallas_call(
    _norm_body,
    grid=(NBLK,),
    in_specs=[
        pl.BlockSpec((1, 1, D), lambda i: (i // BPB, 0, 0)),
        pl.BlockSpec((1, BLK, T), lambda i: (i, 0, 0)),
        pl.BlockSpec((BLK, T * D), lambda i: (i, 0)),
        pl.BlockSpec((2, D), lambda i: (0, 0)),
        pl.BlockSpec((D, D), lambda i: (0, 0)),
        pl.BlockSpec((1, D), lambda i: (0, 0)),
        pl.BlockSpec((D, D), lambda i: (0, 0)),
        pl.BlockSpec((1, D), lambda i: (0, 0)),
        pl.BlockSpec((1, T * D), lambda i: (0, 0)),
        pl.BlockSpec((1, T * D), lambda i: (0, 0)),
    ],
    out_specs=pl.BlockSpec((BLK, T * D), lambda i: (i, 0)),
    out_shape=jax.ShapeDtypeStruct((B * L, T * D), jnp.float32),
)


def kernel(input_ids, token_type_ids, timesteps, word_emb, type_emb,
           W1, b1, W2, b2, gamma, beta):
    idx = jnp.transpose(input_ids.astype(jnp.int32), (0, 2, 1)).reshape(NW, NCHUNK, CH)
    gathered = _build_sc_gather()(word_emb, idx).reshape(B * L, T * D)
    tt3 = jnp.transpose(token_type_ids, (0, 2, 1)).astype(jnp.float32).reshape(NBLK, BLK, T)
    ts_b = jnp.broadcast_to(
        timesteps.astype(jnp.float32)[:, None], (B, D)).reshape(B, 1, D)
    out = _norm(ts_b, tt3, gathered, type_emb, W1.T, b1.reshape(1, D),
                W2.T, b2.reshape(1, D), gamma.reshape(1, T * D),
                beta.reshape(1, T * D))
    return out.reshape(B, L, T * D)


# speedup vs baseline: 3.0080x; 1.5602x over previous
import functools
import math

import jax
import jax.numpy as jnp
from jax import lax
from jax.experimental import pallas as pl
from jax.experimental.pallas import tpu as pltpu
from jax.experimental.pallas import tpu_sc as plsc

B, T, L = 4, 2, 4096
V, D = 100000, 128
EPS = 1e-12

NC, NS = 2, 16
NW = NC * NS
N_ROWS = B * T * L
ROWS_PER_W = N_ROWS // NW
OUT_ROWS_PER_W = (B * L) // NW
CH = 128
NCHUNK = ROWS_PER_W // CH
CPH = NCHUNK // T
NBUF = 4
DEPTH = 2

BLK = 2048
NBLK = (B * L) // BLK
BPB = L // BLK


def _sc_gather_body(table_hbm, idx_hbm, out_hbm, idx_v, *bufs_and_sems):
    bufs = bufs_and_sems[:NBUF]
    gsems = bufs_and_sems[NBUF:2 * NBUF]
    ssems = bufs_and_sems[2 * NBUF:3 * NBUF]
    wid = lax.axis_index("s") * NC + lax.axis_index("c")
    row_base = wid * OUT_ROWS_PER_W
    pltpu.sync_copy(idx_hbm.at[wid], idx_v)
    ghandles = [None] * NBUF
    shandles = [None] * NBUF
    for cc in range(NCHUNK + DEPTH):
        if cc < NCHUNK:
            bi = cc % NBUF
            if cc >= NBUF:
                shandles[bi].wait()
            ghandles[bi] = pltpu.async_copy(
                table_hbm.at[idx_v.at[cc]], bufs[bi], gsems[bi])
        d = cc - DEPTH
        if 0 <= d < NCHUNK:
            bj = d % NBUF
            ghandles[bj].wait()
            h, c = d // CPH, d % CPH
            shandles[bj] = pltpu.async_copy(
                bufs[bj],
                out_hbm.at[pl.ds(row_base + c * CH, CH), pl.ds(h * D, D)],
                ssems[bj])
    for d in range(NCHUNK - NBUF, NCHUNK):
        shandles[d % NBUF].wait()


@functools.cache
def _build_sc_gather():
    return pl.kernel(
        _sc_gather_body,
        mesh=plsc.VectorSubcoreMesh(core_axis_name="c", subcore_axis_name="s"),
        out_type=jax.ShapeDtypeStruct((B * L, T * D), jnp.float32),
        scratch_types=[
            pltpu.VMEM((NCHUNK, CH), jnp.int32),
        ] + [pltpu.VMEM((CH, D), jnp.float32) for _ in range(NBUF)]
          + [pltpu.SemaphoreType.DMA for _ in range(2 * NBUF)],
    )


def _norm_body(ts_ref, tt_ref, g_ref, type_ref, w1_ref, b1_ref, w2_ref,
               b2_ref, gamma_ref, beta_ref, out_ref):
    t_val = ts_ref[0]
    col = lax.broadcasted_iota(jnp.int32, (1, D), 1)
    is_cos = col < (D // 2)
    k = jnp.where(is_cos, col, col - D // 2).astype(jnp.float32)
    freqs = jnp.exp((-math.log(10000.0) / (D // 2)) * k)
    args = t_val * freqs
    te = jnp.where(is_cos, jnp.cos(args), jnp.sin(args))
    h = jnp.dot(te, w1_ref[...], preferred_element_type=jnp.float32) + b1_ref[...]
    h = h * jax.nn.sigmoid(h)
    trow = jnp.dot(h, w2_ref[...], preferred_element_type=jnp.float32) + b2_ref[...]

    ty0 = type_ref[0:1, :]
    dty = type_ref[1:2, :] - ty0
    base_row = ty0 + trow
    g = g_ref[...]
    tt = tt_ref[0]
    half0 = g[:, :D] + base_row + tt[:, 0:1] * dty
    half1 = g[:, D:] + base_row + tt[:, 1:2] * dty
    emb = jnp.concatenate([half0, half1], axis=-1)
    mu = jnp.mean(emb, axis=-1, keepdims=True)
    dev = emb - mu
    var = jnp.mean(dev * dev, axis=-1, keepdims=True)
    out_ref[...] = dev * lax.rsqrt(var + EPS) * gamma_ref[...] + beta_ref[...]


_norm = pl.pallas_call(
    _norm_body,
    grid=(NBLK,),
    in_specs=[
        pl.BlockSpec((1, 1, D), lambda i: (i // BPB, 0, 0)),
        pl.BlockSpec((1, BLK, T), lambda i: (i, 0, 0)),
        pl.BlockSpec((BLK, T * D), lambda i: (i, 0)),
        pl.BlockSpec((2, D), lambda i: (0, 0)),
        pl.BlockSpec((D, D), lambda i: (0, 0)),
        pl.BlockSpec((1, D), lambda i: (0, 0)),
        pl.BlockSpec((D, D), lambda i: (0, 0)),
        pl.BlockSpec((1, D), lambda i: (0, 0)),
        pl.BlockSpec((1, T * D), lambda i: (0, 0)),
        pl.BlockSpec((1, T * D), lambda i: (0, 0)),
    ],
    out_specs=pl.BlockSpec((BLK, T * D), lambda i: (i, 0)),
    out_shape=jax.ShapeDtypeStruct((B * L, T * D), jnp.float32),
)


def kernel(input_ids, token_type_ids, timesteps, word_emb, type_emb,
           W1, b1, W2, b2, gamma, beta):
    idsT = jnp.transpose(input_ids.astype(jnp.int32), (0, 2, 1)).reshape(B * L, T)
    idx = idsT.reshape(NW, CPH, CH, T).transpose(0, 3, 1, 2).reshape(NW, NCHUNK, CH)
    gathered = _build_sc_gather()(word_emb, idx)
    tt3 = jnp.transpose(token_type_ids, (0, 2, 1)).astype(jnp.float32).reshape(NBLK, BLK, T)
    ts_b = jnp.broadcast_to(
        timesteps.astype(jnp.float32)[:, None], (B, D)).reshape(B, 1, D)
    out = _norm(ts_b, tt3, gathered, type_emb, W1.T, b1.reshape(1, D),
                W2.T, b2.reshape(1, D), gamma.reshape(1, T * D),
                beta.reshape(1, T * D))
    return out.reshape(B, L, T * D)
